# baseline (speedup 1.0000x reference)
import functools

import jax
import jax.numpy as jnp
from jax.experimental import pallas as pl
from jax.experimental.pallas import tpu as pltpu

HIDDEN = 1024
INTER = 2816
K_ACTIVE = 704
ROW_TILE = 256
N_TILES = 16
CHUNK = 128
N_BITS = 22


def _fused_kernel(x_ref, wg_ref, wu_ref, wd_ref, out_ref,
                  za_ref, aza_ref, zb_ref, azb_ref):
    i = pl.program_id(0)
    nt = (((1,), (1,)), ((), ()))

    def stage(zc_ref, azc_ref, zp_ref, azp_ref):
        xb = x_ref[...].astype(jnp.bfloat16)

        def body(j, res):
            col = j * CHUNK
            wg_c = wg_ref[pl.ds(col, CHUNK), :]
            wu_c = wu_ref[pl.ds(col, CHUNK), :]
            g = jax.lax.dot_general(xb, wg_c, nt, preferred_element_type=jnp.float32)
            u = jax.lax.dot_general(xb, wu_c, nt, preferred_element_type=jnp.float32)
            zc = (g * jax.nn.sigmoid(g)) * u
            zc_ref[:, pl.ds(col, CHUNK)] = zc
            azc_ref[:, pl.ds(col, CHUNK)] = jnp.abs(zc)

            cand = res | (jnp.int32(1) << (jnp.int32(30) - j))
            candf = jax.lax.bitcast_convert_type(cand, jnp.float32)
            ones = jnp.where(azp_ref[...] >= candf, 1.0, 0.0)
            cnt = jnp.sum(ones, axis=1, keepdims=True)
            return jnp.where(cnt >= float(K_ACTIVE), cand, res)

        res0 = jnp.zeros((ROW_TILE, 1), jnp.int32)
        res = jax.lax.fori_loop(0, N_BITS, body, res0)

        thresh = jax.lax.bitcast_convert_type(res, jnp.float32)
        zm = jnp.where(azp_ref[...] >= thresh, zp_ref[...], 0.0).astype(jnp.bfloat16)
        out_ref[...] = jax.lax.dot_general(
            zm, wd_ref[...], nt, preferred_element_type=jnp.float32
        )

    @pl.when(i % 2 == 0)
    def _():
        stage(za_ref, aza_ref, zb_ref, azb_ref)

    @pl.when(i % 2 == 1)
    def _():
        stage(zb_ref, azb_ref, za_ref, aza_ref)


@jax.jit
def kernel(x, Wg, Wu, Wd):
    B, S, H = x.shape
    rows = B * S
    xf = x.reshape(rows, H)

    out = pl.pallas_call(
        _fused_kernel,
        grid=(N_TILES + 1,),
        in_specs=[
            pl.BlockSpec((ROW_TILE, HIDDEN), lambda i: (jnp.minimum(i, N_TILES - 1), 0)),
            pl.BlockSpec((INTER, HIDDEN), lambda i: (0, 0)),
            pl.BlockSpec((INTER, HIDDEN), lambda i: (0, 0)),
            pl.BlockSpec((HIDDEN, INTER), lambda i: (0, 0)),
        ],
        out_specs=pl.BlockSpec(
            (ROW_TILE, HIDDEN), lambda i: (jnp.maximum(i - 1, 0), 0)
        ),
        out_shape=jax.ShapeDtypeStruct((rows, HIDDEN), jnp.float32),
        scratch_shapes=[
            pltpu.VMEM((ROW_TILE, INTER), jnp.float32),
            pltpu.VMEM((ROW_TILE, INTER), jnp.float32),
            pltpu.VMEM((ROW_TILE, INTER), jnp.float32),
            pltpu.VMEM((ROW_TILE, INTER), jnp.float32),
        ],
        compiler_params=pltpu.CompilerParams(
            dimension_semantics=("arbitrary",),
        ),
    )(
        xf,
        Wg.astype(jnp.bfloat16),
        Wu.astype(jnp.bfloat16),
        Wd.astype(jnp.bfloat16),
    )
    return out.reshape(B, S, H)


# speedup vs baseline: 1.3174x; 1.3174x over previous
import functools

import jax
import jax.numpy as jnp
from jax.experimental import pallas as pl
from jax.experimental.pallas import tpu as pltpu

HIDDEN = 1024
INTER = 2816
K_ACTIVE = 704
ROW_TILE = 256
HALF = ROW_TILE // 2
N_BITS = 22


def _fused_kernel(x_ref, wg_ref, wu_ref, wd_ref, out_ref, az_ref):
    x = x_ref[...].astype(jnp.bfloat16)

    nt = (((1,), (1,)), ((), ()))
    g = jax.lax.dot_general(x, wg_ref[...], nt, preferred_element_type=jnp.float32)
    u = jax.lax.dot_general(x, wu_ref[...], nt, preferred_element_type=jnp.float32)
    z = (g * jax.nn.sigmoid(g)) * u
    az_ref[...] = jnp.abs(z)

    def body(j, carry):
        res_a, res_b = carry
        bit = jnp.int32(1) << (jnp.int32(30) - j)

        cand_a = res_a | bit
        cf_a = jax.lax.bitcast_convert_type(cand_a, jnp.float32)
        ones_a = jnp.where(az_ref[:HALF, :] >= cf_a, 1.0, 0.0)
        cnt_a = jnp.sum(ones_a, axis=1, keepdims=True)

        cand_b = res_b | bit
        cf_b = jax.lax.bitcast_convert_type(cand_b, jnp.float32)
        ones_b = jnp.where(az_ref[HALF:, :] >= cf_b, 1.0, 0.0)
        cnt_b = jnp.sum(ones_b, axis=1, keepdims=True)

        return (
            jnp.where(cnt_a >= float(K_ACTIVE), cand_a, res_a),
            jnp.where(cnt_b >= float(K_ACTIVE), cand_b, res_b),
        )

    res0 = jnp.zeros((HALF, 1), jnp.int32)
    res_a, res_b = jax.lax.fori_loop(0, N_BITS, body, (res0, res0))
    res = jnp.concatenate([res_a, res_b], axis=0)

    thresh = jax.lax.bitcast_convert_type(res, jnp.float32)
    zm = jnp.where(az_ref[...] >= thresh, z, 0.0).astype(jnp.bfloat16)
    out_ref[...] = jax.lax.dot_general(
        zm, wd_ref[...], nt, preferred_element_type=jnp.float32
    )


@jax.jit
def kernel(x, Wg, Wu, Wd):
    B, S, H = x.shape
    rows = B * S
    xf = x.reshape(rows, H)

    out = pl.pallas_call(
        _fused_kernel,
        grid=(rows // ROW_TILE,),
        in_specs=[
            pl.BlockSpec((ROW_TILE, HIDDEN), lambda i: (i, 0)),
            pl.BlockSpec((INTER, HIDDEN), lambda i: (0, 0)),
            pl.BlockSpec((INTER, HIDDEN), lambda i: (0, 0)),
            pl.BlockSpec((HIDDEN, INTER), lambda i: (0, 0)),
        ],
        out_specs=pl.BlockSpec((ROW_TILE, HIDDEN), lambda i: (i, 0)),
        out_shape=jax.ShapeDtypeStruct((rows, HIDDEN), jnp.float32),
        scratch_shapes=[pltpu.VMEM((ROW_TILE, INTER), jnp.float32)],
        compiler_params=pltpu.CompilerParams(
            dimension_semantics=("arbitrary",),
        ),
    )(
        xf,
        Wg.astype(jnp.bfloat16),
        Wu.astype(jnp.bfloat16),
        Wd.astype(jnp.bfloat16),
    )
    return out.reshape(B, S, H)
